# Initial kernel scaffold; baseline (speedup 1.0000x reference)
import jax
import jax.numpy as jnp
from jax.experimental import pallas as pl


def kernel(hidden_states, Wq, Wk, Wv, Wo):
    raise NotImplementedError("write your pallas kernel here")


# speedup vs baseline: 70.7588x; 70.7588x over previous
import jax
import jax.numpy as jnp
import numpy as np
from jax.experimental import pallas as pl
from jax.experimental.pallas import tpu as pltpu

S, D = 2048, 768
H, KVH, HD = 12, 4, 64
REP = H // KVH
SCALE = 1.0 / np.sqrt(HD)

QB = 256
SLAB = QB + 64
NLAND = S // 64
NT = S // QB


def _proj_kernel(hs_ref, wq_ref, wk_ref, wv_ref,
                 q_ref, k_ref, v_ref, lk_ref, lv_ref):
    hs = hs_ref[...]
    q_ref[...] = jnp.dot(hs, wq_ref[...], preferred_element_type=jnp.float32)
    k = jnp.dot(hs, wk_ref[...], preferred_element_type=jnp.float32)
    v = jnp.dot(hs, wv_ref[...], preferred_element_type=jnp.float32)
    k_ref[...] = k
    v_ref[...] = v
    lk_ref[...] = jnp.concatenate(
        [k[i:i + 1] for i in range(0, QB, 64)], axis=0)[None]
    lv_ref[...] = jnp.concatenate(
        [v[i:i + 1] for i in range(0, QB, 64)], axis=0)[None]


def _attn_kernel(q_ref, kp_ref, vp_ref, lk_ref, lv_ref, wo_ref, out_ref):
    t = pl.program_id(0)
    q = q_ref[...]
    slab_k = kp_ref[pl.ds(t * QB, SLAB), :]
    slab_v = vp_ref[pl.ds(t * QB, SLAB), :]
    lk = lk_ref[...]
    lv = lv_ref[...]

    r_w = jax.lax.broadcasted_iota(jnp.int32, (QB, SLAB), 0)
    j_w = jax.lax.broadcasted_iota(jnp.int32, (QB, SLAB), 1)
    win_mask = (j_w > r_w) & (j_w <= r_w + 64) & (j_w + QB * t >= 64)
    r_l = jax.lax.broadcasted_iota(jnp.int32, (QB, NLAND), 0)
    m_l = jax.lax.broadcasted_iota(jnp.int32, (QB, NLAND), 1)
    land_mask = m_l < (QB // 64) * t + r_l // 64

    ctxs = []
    for h in range(H):
        g = h // REP
        qh = q[:, h * HD:(h + 1) * HD]
        kg = slab_k[:, g * HD:(g + 1) * HD]
        lkg = lk[:, g * HD:(g + 1) * HD]
        s1 = jax.lax.dot_general(qh, kg, (((1,), (1,)), ((), ())),
                                 preferred_element_type=jnp.float32) * SCALE
        s2 = jax.lax.dot_general(qh, lkg, (((1,), (1,)), ((), ())),
                                 preferred_element_type=jnp.float32) * SCALE
        s = jnp.concatenate([jnp.where(win_mask, s1, -1e30),
                             jnp.where(land_mask, s2, -1e30)], axis=1)
        mx = jnp.max(s, axis=1, keepdims=True)
        p = jnp.exp(s - mx)
        p = p / jnp.sum(p, axis=1, keepdims=True)
        vcat = jnp.concatenate([slab_v[:, g * HD:(g + 1) * HD],
                                lv[:, g * HD:(g + 1) * HD]], axis=0)
        ctxs.append(jnp.dot(p, vcat, preferred_element_type=jnp.float32))
    ctx = jnp.concatenate(ctxs, axis=1)
    out_ref[...] = jnp.dot(ctx, wo_ref[...], preferred_element_type=jnp.float32)


def kernel(hidden_states, Wq, Wk, Wv, Wo):
    b = hidden_states.shape[0]
    hs = hidden_states.reshape(S, D)

    q, k, v, lk, lv = pl.pallas_call(
        _proj_kernel,
        grid=(NT,),
        in_specs=[
            pl.BlockSpec((QB, D), lambda t: (t, 0)),
            pl.BlockSpec((D, H * HD), lambda t: (0, 0)),
            pl.BlockSpec((D, KVH * HD), lambda t: (0, 0)),
            pl.BlockSpec((D, KVH * HD), lambda t: (0, 0)),
        ],
        out_specs=[
            pl.BlockSpec((QB, H * HD), lambda t: (t, 0)),
            pl.BlockSpec((QB, KVH * HD), lambda t: (t, 0)),
            pl.BlockSpec((QB, KVH * HD), lambda t: (t, 0)),
            pl.BlockSpec((1, QB // 64, KVH * HD), lambda t: (t, 0, 0)),
            pl.BlockSpec((1, QB // 64, KVH * HD), lambda t: (t, 0, 0)),
        ],
        out_shape=[
            jax.ShapeDtypeStruct((S, H * HD), jnp.float32),
            jax.ShapeDtypeStruct((S, KVH * HD), jnp.float32),
            jax.ShapeDtypeStruct((S, KVH * HD), jnp.float32),
            jax.ShapeDtypeStruct((NT, QB // 64, KVH * HD), jnp.float32),
            jax.ShapeDtypeStruct((NT, QB // 64, KVH * HD), jnp.float32),
        ],
        compiler_params=pltpu.CompilerParams(
            dimension_semantics=("parallel",)),
    )(hs, Wq, Wk, Wv)

    kp = jnp.pad(k, ((64, 0), (0, 0)))
    vp = jnp.pad(v, ((64, 0), (0, 0)))
    lk = lk.reshape(NLAND, KVH * HD)
    lv = lv.reshape(NLAND, KVH * HD)

    out = pl.pallas_call(
        _attn_kernel,
        grid=(NT,),
        in_specs=[
            pl.BlockSpec((QB, H * HD), lambda t: (t, 0)),
            pl.BlockSpec((S + 64, KVH * HD), lambda t: (0, 0)),
            pl.BlockSpec((S + 64, KVH * HD), lambda t: (0, 0)),
            pl.BlockSpec((NLAND, KVH * HD), lambda t: (0, 0)),
            pl.BlockSpec((NLAND, KVH * HD), lambda t: (0, 0)),
            pl.BlockSpec((H * HD, D), lambda t: (0, 0)),
        ],
        out_specs=pl.BlockSpec((QB, D), lambda t: (t, 0)),
        out_shape=jax.ShapeDtypeStruct((S, D), jnp.float32),
        compiler_params=pltpu.CompilerParams(
            dimension_semantics=("parallel",)),
    )(q, kp, vp, lk, lv, Wo)

    return out.reshape(b, S, D)
